# Initial kernel scaffold; baseline (speedup 1.0000x reference)
import jax
import jax.numpy as jnp
from jax.experimental import pallas as pl


def kernel(input_GRU, input_linear, init_hidden, w_ih, w_hh, b_ih, b_n, lin_bias):
    raise NotImplementedError("write your pallas kernel here")


# speedup vs baseline: 11.5337x; 11.5337x over previous
import jax
import jax.numpy as jnp
from jax.experimental import pallas as pl
from jax.experimental.pallas import tpu as pltpu

IN = 96
H = 128
LIN = 32
CHUNK = 512


def _gru_body(x_ref, xl_ref, wih_ref, whh_ref, bih_ref, bn_ref, h0_ref, lb_ref,
              out_ref, ig_ref, hs_ref, hc_ref):
    i = pl.program_id(0)

    @pl.when(i == 0)
    def _init():
        hc_ref[...] = h0_ref[...]

    ig_ref[...] = (
        jnp.dot(x_ref[...], wih_ref[...], preferred_element_type=jnp.float32)
        + bih_ref[...]
    )

    whh = whh_ref[...]
    bn = bn_ref[...]

    def step(t, h):
        ig = ig_ref[pl.ds(t, 1), :]
        hg = jnp.dot(h, whh, preferred_element_type=jnp.float32)
        r = jax.nn.sigmoid(ig[:, 0:H] + hg[:, 0:H])
        z = jax.nn.sigmoid(ig[:, H:2 * H] + hg[:, H:2 * H])
        n = jnp.tanh(ig[:, 2 * H:3 * H] + r * (hg[:, 2 * H:3 * H] + bn))
        h = n + z * (h - n)
        hs_ref[pl.ds(t, 1), :] = h
        return h

    h = jax.lax.fori_loop(0, CHUNK, step, hc_ref[...], unroll=8)
    hc_ref[...] = h

    out_ref[...] = (
        jnp.sum(hs_ref[:, 0:LIN] * xl_ref[...], axis=1, keepdims=True)
        + lb_ref[...]
    )


def kernel(input_GRU, input_linear, init_hidden, w_ih, w_hh, b_ih, b_n, lin_bias):
    T = input_GRU.shape[0]
    grid = T // CHUNK

    out = pl.pallas_call(
        _gru_body,
        grid=(grid,),
        in_specs=[
            pl.BlockSpec((CHUNK, IN), lambda i: (i, 0)),
            pl.BlockSpec((CHUNK, LIN), lambda i: (i, 0)),
            pl.BlockSpec((IN, 3 * H), lambda i: (0, 0)),
            pl.BlockSpec((H, 3 * H), lambda i: (0, 0)),
            pl.BlockSpec((1, 3 * H), lambda i: (0, 0)),
            pl.BlockSpec((1, H), lambda i: (0, 0)),
            pl.BlockSpec((1, H), lambda i: (0, 0)),
            pl.BlockSpec((1, 1), lambda i: (0, 0)),
        ],
        out_specs=pl.BlockSpec((CHUNK, 1), lambda i: (i, 0)),
        out_shape=jax.ShapeDtypeStruct((T, 1), jnp.float32),
        scratch_shapes=[
            pltpu.VMEM((CHUNK, 3 * H), jnp.float32),
            pltpu.VMEM((CHUNK, H), jnp.float32),
            pltpu.VMEM((1, H), jnp.float32),
        ],
        compiler_params=pltpu.CompilerParams(
            dimension_semantics=("arbitrary",),
        ),
    )(
        input_GRU,
        input_linear,
        w_ih.T,
        w_hh.T,
        b_ih[None, :],
        b_n[None, :],
        init_hidden[None, :],
        lin_bias[None, :],
    )
    return out


# speedup vs baseline: 11.7047x; 1.0148x over previous
import jax
import jax.numpy as jnp
from jax.experimental import pallas as pl
from jax.experimental.pallas import tpu as pltpu

IN = 96
H = 128
LIN = 32
CHUNK = 512


def _gru_body(x_ref, xl_ref, wih_ref, whh_ref, bih_ref, bn_ref, h0_ref, lb_ref,
              out_ref, ig_ref, hs_ref, hc_ref):
    i = pl.program_id(0)

    @pl.when(i == 0)
    def _init():
        hc_ref[...] = h0_ref[...]

    ig_ref[...] = (
        jnp.dot(x_ref[...], wih_ref[...], preferred_element_type=jnp.float32)
        + bih_ref[...]
    )

    whh = whh_ref[...]
    bn = bn_ref[...]

    def step(t, h):
        ig = ig_ref[pl.ds(t, 1), :]
        hg = jnp.dot(h, whh, preferred_element_type=jnp.float32)
        r = jax.nn.sigmoid(ig[:, 0:H] + hg[:, 0:H])
        z = jax.nn.sigmoid(ig[:, H:2 * H] + hg[:, H:2 * H])
        n = jnp.tanh(ig[:, 2 * H:3 * H] + r * (hg[:, 2 * H:3 * H] + bn))
        h = (1.0 - z) * n + z * h
        hs_ref[pl.ds(t, 1), :] = h
        return h

    h = jax.lax.fori_loop(0, CHUNK, step, hc_ref[...], unroll=16)
    hc_ref[...] = h

    out_ref[...] = (
        jnp.sum(hs_ref[:, 0:LIN] * xl_ref[...], axis=1, keepdims=True)
        + lb_ref[...]
    )


def kernel(input_GRU, input_linear, init_hidden, w_ih, w_hh, b_ih, b_n, lin_bias):
    T = input_GRU.shape[0]
    grid = T // CHUNK

    out = pl.pallas_call(
        _gru_body,
        grid=(grid,),
        in_specs=[
            pl.BlockSpec((CHUNK, IN), lambda i: (i, 0)),
            pl.BlockSpec((CHUNK, LIN), lambda i: (i, 0)),
            pl.BlockSpec((IN, 3 * H), lambda i: (0, 0)),
            pl.BlockSpec((H, 3 * H), lambda i: (0, 0)),
            pl.BlockSpec((1, 3 * H), lambda i: (0, 0)),
            pl.BlockSpec((1, H), lambda i: (0, 0)),
            pl.BlockSpec((1, H), lambda i: (0, 0)),
            pl.BlockSpec((1, 1), lambda i: (0, 0)),
        ],
        out_specs=pl.BlockSpec((CHUNK, 1), lambda i: (i, 0)),
        out_shape=jax.ShapeDtypeStruct((T, 1), jnp.float32),
        scratch_shapes=[
            pltpu.VMEM((CHUNK, 3 * H), jnp.float32),
            pltpu.VMEM((CHUNK, H), jnp.float32),
            pltpu.VMEM((1, H), jnp.float32),
        ],
        compiler_params=pltpu.CompilerParams(
            dimension_semantics=("arbitrary",),
        ),
    )(
        input_GRU,
        input_linear,
        w_ih.T,
        w_hh.T,
        b_ih[None, :],
        b_n[None, :],
        init_hidden[None, :],
        lin_bias[None, :],
    )
    return out


# speedup vs baseline: 12.3984x; 1.0593x over previous
import jax
import jax.numpy as jnp
from jax.experimental import pallas as pl
from jax.experimental.pallas import tpu as pltpu

IN = 96
H = 128
LIN = 32
CHUNK = 512


def _gru_body(x_ref, xl_ref, wih_ref, whh_ref, bih_ref, bn_ref, h0_ref, lb_ref,
              out_ref, ig_ref, hs_ref, hc_ref):
    i = pl.program_id(0)

    @pl.when(i == 0)
    def _init():
        hc_ref[...] = h0_ref[...]

    ig_ref[...] = (
        jnp.dot(x_ref[...], wih_ref[...], preferred_element_type=jnp.float32)
        + bih_ref[...]
    )

    whh = whh_ref[...]
    bnh = bn_ref[...]

    def step(t, h):
        ig = ig_ref[pl.ds(t, 1), :]
        hg = jnp.dot(h, whh, preferred_element_type=jnp.float32)
        tr = jnp.tanh(ig[:, 0:H] + hg[:, 0:H])
        tz = jnp.tanh(ig[:, H:2 * H] + hg[:, H:2 * H])
        m2 = hg[:, 2 * H:3 * H] + bnh
        c1 = ig[:, 2 * H:3 * H] + m2
        n = jnp.tanh(tr * m2 + c1)
        a = 0.5 - 0.5 * tz
        c = (0.5 + 0.5 * tz) * h
        h = a * n + c
        hs_ref[pl.ds(t, 1), :] = h
        return h

    h = jax.lax.fori_loop(0, CHUNK, step, hc_ref[...], unroll=16)
    hc_ref[...] = h

    out_ref[...] = (
        jnp.sum(hs_ref[:, 0:LIN] * xl_ref[...], axis=1, keepdims=True)
        + lb_ref[...]
    )


def kernel(input_GRU, input_linear, init_hidden, w_ih, w_hh, b_ih, b_n, lin_bias):
    T = input_GRU.shape[0]
    grid = T // CHUNK

    half = jnp.float32(0.5)
    col_scale = jnp.concatenate(
        [jnp.full((2 * H,), half), jnp.ones((H,), jnp.float32)]
    )
    wih_s = w_ih.T * col_scale[None, :]
    whh_s = w_hh.T * half
    bih_s = (b_ih * col_scale)[None, :]
    bn_s = (b_n * half)[None, :]

    out = pl.pallas_call(
        _gru_body,
        grid=(grid,),
        in_specs=[
            pl.BlockSpec((CHUNK, IN), lambda i: (i, 0)),
            pl.BlockSpec((CHUNK, LIN), lambda i: (i, 0)),
            pl.BlockSpec((IN, 3 * H), lambda i: (0, 0)),
            pl.BlockSpec((H, 3 * H), lambda i: (0, 0)),
            pl.BlockSpec((1, 3 * H), lambda i: (0, 0)),
            pl.BlockSpec((1, H), lambda i: (0, 0)),
            pl.BlockSpec((1, H), lambda i: (0, 0)),
            pl.BlockSpec((1, 1), lambda i: (0, 0)),
        ],
        out_specs=pl.BlockSpec((CHUNK, 1), lambda i: (i, 0)),
        out_shape=jax.ShapeDtypeStruct((T, 1), jnp.float32),
        scratch_shapes=[
            pltpu.VMEM((CHUNK, 3 * H), jnp.float32),
            pltpu.VMEM((CHUNK, H), jnp.float32),
            pltpu.VMEM((1, H), jnp.float32),
        ],
        compiler_params=pltpu.CompilerParams(
            dimension_semantics=("arbitrary",),
        ),
    )(
        input_GRU,
        input_linear,
        wih_s,
        whh_s,
        bih_s,
        bn_s,
        init_hidden[None, :],
        lin_bias[None, :],
    )
    return out


# speedup vs baseline: 12.4853x; 1.0070x over previous
import jax
import jax.numpy as jnp
from jax.experimental import pallas as pl
from jax.experimental.pallas import tpu as pltpu

IN = 96
H = 128
LIN = 32
CHUNK = 1024


def _gru_body(x_ref, xl_ref, wih_ref, whh_ref, bih_ref, bn_ref, h0_ref, lb_ref,
              out_ref, ig_ref, hs_ref, hc_ref):
    i = pl.program_id(0)

    @pl.when(i == 0)
    def _init():
        hc_ref[...] = h0_ref[...]

    ig_ref[...] = (
        jnp.dot(x_ref[...], wih_ref[...], preferred_element_type=jnp.float32)
        + bih_ref[...]
    )

    whh = whh_ref[...]
    bnh = bn_ref[...]

    def step(t, h):
        ig = ig_ref[pl.ds(t, 1), :]
        hg = jnp.dot(h, whh, preferred_element_type=jnp.float32)
        tr = jnp.tanh(ig[:, 0:H] + hg[:, 0:H])
        tz = jnp.tanh(ig[:, H:2 * H] + hg[:, H:2 * H])
        m2 = hg[:, 2 * H:3 * H] + bnh
        c1 = ig[:, 2 * H:3 * H] + m2
        n = jnp.tanh(tr * m2 + c1)
        a = 0.5 - 0.5 * tz
        c = (0.5 + 0.5 * tz) * h
        h = a * n + c
        hs_ref[pl.ds(t, 1), :] = h
        return h

    h = jax.lax.fori_loop(0, CHUNK, step, hc_ref[...], unroll=32)
    hc_ref[...] = h

    out_ref[...] = (
        jnp.sum(hs_ref[:, 0:LIN] * xl_ref[...], axis=1, keepdims=True)
        + lb_ref[...]
    )


def kernel(input_GRU, input_linear, init_hidden, w_ih, w_hh, b_ih, b_n, lin_bias):
    T = input_GRU.shape[0]
    grid = T // CHUNK

    half = jnp.float32(0.5)
    col_scale = jnp.concatenate(
        [jnp.full((2 * H,), half), jnp.ones((H,), jnp.float32)]
    )
    wih_s = w_ih.T * col_scale[None, :]
    whh_s = w_hh.T * half
    bih_s = (b_ih * col_scale)[None, :]
    bn_s = (b_n * half)[None, :]

    out = pl.pallas_call(
        _gru_body,
        grid=(grid,),
        in_specs=[
            pl.BlockSpec((CHUNK, IN), lambda i: (i, 0)),
            pl.BlockSpec((CHUNK, LIN), lambda i: (i, 0)),
            pl.BlockSpec((IN, 3 * H), lambda i: (0, 0)),
            pl.BlockSpec((H, 3 * H), lambda i: (0, 0)),
            pl.BlockSpec((1, 3 * H), lambda i: (0, 0)),
            pl.BlockSpec((1, H), lambda i: (0, 0)),
            pl.BlockSpec((1, H), lambda i: (0, 0)),
            pl.BlockSpec((1, 1), lambda i: (0, 0)),
        ],
        out_specs=pl.BlockSpec((CHUNK, 1), lambda i: (i, 0)),
        out_shape=jax.ShapeDtypeStruct((T, 1), jnp.float32),
        scratch_shapes=[
            pltpu.VMEM((CHUNK, 3 * H), jnp.float32),
            pltpu.VMEM((CHUNK, H), jnp.float32),
            pltpu.VMEM((1, H), jnp.float32),
        ],
        compiler_params=pltpu.CompilerParams(
            dimension_semantics=("arbitrary",),
        ),
    )(
        input_GRU,
        input_linear,
        wih_s,
        whh_s,
        bih_s,
        bn_s,
        init_hidden[None, :],
        lin_bias[None, :],
    )
    return out


# speedup vs baseline: 12.5039x; 1.0015x over previous
import jax
import jax.numpy as jnp
from jax.experimental import pallas as pl
from jax.experimental.pallas import tpu as pltpu

IN = 96
H = 128
LIN = 32
CHUNK = 2048


def _gru_body(x_ref, xl_ref, wih_ref, whh_ref, bih_ref, bn_ref, h0_ref, lb_ref,
              out_ref, ig_ref, hs_ref, hc_ref):
    i = pl.program_id(0)

    @pl.when(i == 0)
    def _init():
        hc_ref[...] = h0_ref[...]

    ig_ref[...] = (
        jnp.dot(x_ref[...], wih_ref[...], preferred_element_type=jnp.float32)
        + bih_ref[...]
    )

    whh = whh_ref[...]
    bnh = bn_ref[...]

    def step(t, h):
        ig = ig_ref[pl.ds(t, 1), :]
        hg = jnp.dot(h, whh, preferred_element_type=jnp.float32)
        tr = jnp.tanh(ig[:, 0:H] + hg[:, 0:H])
        tz = jnp.tanh(ig[:, H:2 * H] + hg[:, H:2 * H])
        m2 = hg[:, 2 * H:3 * H] + bnh
        c1 = ig[:, 2 * H:3 * H] + m2
        n = jnp.tanh(tr * m2 + c1)
        a = 0.5 - 0.5 * tz
        c = (0.5 + 0.5 * tz) * h
        h = a * n + c
        hs_ref[pl.ds(t, 1), :] = h
        return h

    h = jax.lax.fori_loop(0, CHUNK, step, hc_ref[...], unroll=32)
    hc_ref[...] = h

    out_ref[...] = (
        jnp.sum(hs_ref[:, 0:LIN] * xl_ref[...], axis=1, keepdims=True)
        + lb_ref[...]
    )


def kernel(input_GRU, input_linear, init_hidden, w_ih, w_hh, b_ih, b_n, lin_bias):
    T = input_GRU.shape[0]
    grid = T // CHUNK

    half = jnp.float32(0.5)
    col_scale = jnp.concatenate(
        [jnp.full((2 * H,), half), jnp.ones((H,), jnp.float32)]
    )
    wih_s = w_ih.T * col_scale[None, :]
    whh_s = w_hh.T * half
    bih_s = (b_ih * col_scale)[None, :]
    bn_s = (b_n * half)[None, :]

    out = pl.pallas_call(
        _gru_body,
        grid=(grid,),
        in_specs=[
            pl.BlockSpec((CHUNK, IN), lambda i: (i, 0)),
            pl.BlockSpec((CHUNK, LIN), lambda i: (i, 0)),
            pl.BlockSpec((IN, 3 * H), lambda i: (0, 0)),
            pl.BlockSpec((H, 3 * H), lambda i: (0, 0)),
            pl.BlockSpec((1, 3 * H), lambda i: (0, 0)),
            pl.BlockSpec((1, H), lambda i: (0, 0)),
            pl.BlockSpec((1, H), lambda i: (0, 0)),
            pl.BlockSpec((1, 1), lambda i: (0, 0)),
        ],
        out_specs=pl.BlockSpec((CHUNK, 1), lambda i: (i, 0)),
        out_shape=jax.ShapeDtypeStruct((T, 1), jnp.float32),
        scratch_shapes=[
            pltpu.VMEM((CHUNK, 3 * H), jnp.float32),
            pltpu.VMEM((CHUNK, H), jnp.float32),
            pltpu.VMEM((1, H), jnp.float32),
        ],
        compiler_params=pltpu.CompilerParams(
            dimension_semantics=("arbitrary",),
        ),
    )(
        input_GRU,
        input_linear,
        wih_s,
        whh_s,
        bih_s,
        bn_s,
        init_hidden[None, :],
        lin_bias[None, :],
    )
    return out


# speedup vs baseline: 12.5223x; 1.0015x over previous
import jax
import jax.numpy as jnp
from jax.experimental import pallas as pl
from jax.experimental.pallas import tpu as pltpu

IN = 96
H = 128
LIN = 32
CHUNK = 2048


def _gru_body(x_ref, xl_ref, wih_ref, whh_ref, bih_ref, bn_ref, h0_ref, lb_ref,
              out_ref, ig_ref, hs_ref, hc_ref):
    i = pl.program_id(0)

    @pl.when(i == 0)
    def _init():
        hc_ref[...] = h0_ref[...]

    ig_ref[...] = (
        jnp.dot(x_ref[...], wih_ref[...], preferred_element_type=jnp.float32)
        + bih_ref[...]
    )

    whh = whh_ref[...]
    bnh = bn_ref[...]

    def step(t, h):
        ig = ig_ref[pl.ds(t, 1), :]
        hg = jnp.dot(h, whh, preferred_element_type=jnp.float32)
        tr = jnp.tanh(ig[:, 0:H] + hg[:, 0:H])
        tz = jnp.tanh(ig[:, H:2 * H] + hg[:, H:2 * H])
        m2 = hg[:, 2 * H:3 * H] + bnh
        c1 = ig[:, 2 * H:3 * H] + m2
        n = jnp.tanh(tr * m2 + c1)
        a = 0.5 - 0.5 * tz
        c = (0.5 + 0.5 * tz) * h
        h = a * n + c
        hs_ref[pl.ds(t, 1), :] = h
        return h

    h = jax.lax.fori_loop(0, CHUNK, step, hc_ref[...], unroll=64)
    hc_ref[...] = h

    out_ref[...] = (
        jnp.sum(hs_ref[:, 0:LIN] * xl_ref[...], axis=1, keepdims=True)
        + lb_ref[...]
    )


def kernel(input_GRU, input_linear, init_hidden, w_ih, w_hh, b_ih, b_n, lin_bias):
    T = input_GRU.shape[0]
    grid = T // CHUNK

    half = jnp.float32(0.5)
    col_scale = jnp.concatenate(
        [jnp.full((2 * H,), half), jnp.ones((H,), jnp.float32)]
    )
    wih_s = w_ih.T * col_scale[None, :]
    whh_s = w_hh.T * half
    bih_s = (b_ih * col_scale)[None, :]
    bn_s = (b_n * half)[None, :]

    out = pl.pallas_call(
        _gru_body,
        grid=(grid,),
        in_specs=[
            pl.BlockSpec((CHUNK, IN), lambda i: (i, 0)),
            pl.BlockSpec((CHUNK, LIN), lambda i: (i, 0)),
            pl.BlockSpec((IN, 3 * H), lambda i: (0, 0)),
            pl.BlockSpec((H, 3 * H), lambda i: (0, 0)),
            pl.BlockSpec((1, 3 * H), lambda i: (0, 0)),
            pl.BlockSpec((1, H), lambda i: (0, 0)),
            pl.BlockSpec((1, H), lambda i: (0, 0)),
            pl.BlockSpec((1, 1), lambda i: (0, 0)),
        ],
        out_specs=pl.BlockSpec((CHUNK, 1), lambda i: (i, 0)),
        out_shape=jax.ShapeDtypeStruct((T, 1), jnp.float32),
        scratch_shapes=[
            pltpu.VMEM((CHUNK, 3 * H), jnp.float32),
            pltpu.VMEM((CHUNK, H), jnp.float32),
            pltpu.VMEM((1, H), jnp.float32),
        ],
        compiler_params=pltpu.CompilerParams(
            dimension_semantics=("arbitrary",),
        ),
    )(
        input_GRU,
        input_linear,
        wih_s,
        whh_s,
        bih_s,
        bn_s,
        init_hidden[None, :],
        lin_bias[None, :],
    )
    return out


# speedup vs baseline: 12.5413x; 1.0015x over previous
import jax
import jax.numpy as jnp
from jax.experimental import pallas as pl
from jax.experimental.pallas import tpu as pltpu

IN = 96
H = 128
LIN = 32
CHUNK = 2048


def _gru_body(x_ref, xl_ref, wih_ref, whh_ref, bih_ref, bn_ref, h0_ref, lb_ref,
              out_ref, ig_ref, hs_ref, hc_ref):
    i = pl.program_id(0)

    @pl.when(i == 0)
    def _init():
        hc_ref[...] = h0_ref[...]

    ig_ref[...] = (
        jnp.dot(x_ref[...], wih_ref[...], preferred_element_type=jnp.float32)
        + bih_ref[...]
    )

    whh = whh_ref[...]
    bnh = bn_ref[...]

    def step(t, h):
        ig = ig_ref[pl.ds(t, 1), :]
        hg = jnp.dot(h, whh, preferred_element_type=jnp.float32)
        tr = jnp.tanh(ig[:, 0:H] + hg[:, 0:H])
        tz = jnp.tanh(ig[:, H:2 * H] + hg[:, H:2 * H])
        m2 = hg[:, 2 * H:3 * H] + bnh
        c1 = ig[:, 2 * H:3 * H] + m2
        n = jnp.tanh(tr * m2 + c1)
        a = 0.5 - 0.5 * tz
        c = (0.5 + 0.5 * tz) * h
        h = a * n + c
        hs_ref[pl.ds(t, 1), :] = h
        return h

    h = jax.lax.fori_loop(0, CHUNK, step, hc_ref[...], unroll=128)
    hc_ref[...] = h

    out_ref[...] = (
        jnp.sum(hs_ref[:, 0:LIN] * xl_ref[...], axis=1, keepdims=True)
        + lb_ref[...]
    )


def kernel(input_GRU, input_linear, init_hidden, w_ih, w_hh, b_ih, b_n, lin_bias):
    T = input_GRU.shape[0]
    grid = T // CHUNK

    half = jnp.float32(0.5)
    col_scale = jnp.concatenate(
        [jnp.full((2 * H,), half), jnp.ones((H,), jnp.float32)]
    )
    wih_s = w_ih.T * col_scale[None, :]
    whh_s = w_hh.T * half
    bih_s = (b_ih * col_scale)[None, :]
    bn_s = (b_n * half)[None, :]

    out = pl.pallas_call(
        _gru_body,
        grid=(grid,),
        in_specs=[
            pl.BlockSpec((CHUNK, IN), lambda i: (i, 0)),
            pl.BlockSpec((CHUNK, LIN), lambda i: (i, 0)),
            pl.BlockSpec((IN, 3 * H), lambda i: (0, 0)),
            pl.BlockSpec((H, 3 * H), lambda i: (0, 0)),
            pl.BlockSpec((1, 3 * H), lambda i: (0, 0)),
            pl.BlockSpec((1, H), lambda i: (0, 0)),
            pl.BlockSpec((1, H), lambda i: (0, 0)),
            pl.BlockSpec((1, 1), lambda i: (0, 0)),
        ],
        out_specs=pl.BlockSpec((CHUNK, 1), lambda i: (i, 0)),
        out_shape=jax.ShapeDtypeStruct((T, 1), jnp.float32),
        scratch_shapes=[
            pltpu.VMEM((CHUNK, 3 * H), jnp.float32),
            pltpu.VMEM((CHUNK, H), jnp.float32),
            pltpu.VMEM((1, H), jnp.float32),
        ],
        compiler_params=pltpu.CompilerParams(
            dimension_semantics=("arbitrary",),
        ),
    )(
        input_GRU,
        input_linear,
        wih_s,
        whh_s,
        bih_s,
        bn_s,
        init_hidden[None, :],
        lin_bias[None, :],
    )
    return out
